# baseline (speedup 1.0000x reference)
import functools

import jax
import jax.numpy as jnp
from jax import lax
from jax.experimental import pallas as pl
from jax.experimental.pallas import tpu as pltpu
from jax.experimental.pallas import tpu_sc as plsc

N = 10000
E = 320000
D_IN = 128
DH = 64

NC = 2
NS = 16
NW = NC * NS
CHUNK = 128
NCHUNK = 80
EPT = NCHUNK * CHUNK
EP = EPT * NW
NP = 10240
ROWS_PER_TILE = NP // NS

_mesh = lambda: plsc.VectorSubcoreMesh(
    core_axis_name="c", subcore_axis_name="s", num_cores=NC, num_subcores=NS)

_Z16 = lambda: jnp.zeros((16,), jnp.float32)

_SC_PARAMS = pltpu.CompilerParams(needs_layout_passes=False)
_SC_PARAMS_NT = pltpu.CompilerParams(needs_layout_passes=False,
                                     use_tc_tiling_on_sc=False)


@functools.partial(
    pl.kernel,
    out_type=jax.ShapeDtypeStruct((NW, NP), jnp.float32),
    mesh=_mesh(),
    scratch_types=[pltpu.VMEM((EPT,), jnp.int32),
                   pltpu.VMEM((NP,), jnp.float32)],
    compiler_params=_SC_PARAMS,
)
def _deg_call(dst_hbm, out_hbm, didx_v, acc_v):
    cid = lax.axis_index("c")
    sid = lax.axis_index("s")
    wid = sid * NC + cid
    pltpu.sync_copy(dst_hbm.at[pl.ds(wid * EPT, EPT)], didx_v)
    z = _Z16()

    def zero(j, carry):
        acc_v[pl.ds(j * 16, 16)] = z
        return carry
    lax.fori_loop(0, NP // 16, zero, 0)

    ones = jnp.ones((16,), jnp.float32)

    def body(j, carry):
        idx = didx_v[pl.ds(j * 16, 16)]
        plsc.addupdate_scatter(acc_v, [idx], ones)
        return carry
    lax.fori_loop(0, EPT // 16, body, 0)
    pltpu.sync_copy(acc_v, out_hbm.at[wid])


@functools.partial(
    pl.kernel,
    out_type=jax.ShapeDtypeStruct((NC, NP, DH), jnp.float32),
    mesh=_mesh(),
    scratch_types=[pltpu.VMEM((NCHUNK, CHUNK), jnp.int32),
                   pltpu.VMEM((NCHUNK, CHUNK), jnp.int32),
                   pltpu.VMEM((CHUNK, DH), jnp.float32),
                   pltpu.VMEM((CHUNK, DH), jnp.float32),
                   pltpu.VMEM_SHARED((NP, DH), jnp.float32),
                   pltpu.VMEM_SHARED((NP, DH), jnp.float32),
                   pltpu.SemaphoreType.DMA,
                   pltpu.SemaphoreType.DMA],
    compiler_params=_SC_PARAMS_NT,
)
def _edge64_call(src_hbm, dst_hbm, m_hbm, out_hbm, sidx_v, didx_v, rows_a,
                 rows_b, acc_sh, m_sp, sem_a, sem_b):
    cid = lax.axis_index("c")
    sid = lax.axis_index("s")
    wid = sid * NC + cid
    pltpu.sync_copy(src_hbm.at[wid], sidx_v)
    pltpu.sync_copy(dst_hbm.at[wid], didx_v)
    pltpu.sync_copy(m_hbm.at[pl.ds(sid * ROWS_PER_TILE, ROWS_PER_TILE)],
                    m_sp.at[pl.ds(sid * ROWS_PER_TILE, ROWS_PER_TILE)])

    z = _Z16()

    def zrow(r, carry):
        for c4 in range(DH // 16):
            rows_a[r, pl.ds(c4 * 16, 16)] = z
        return carry
    lax.fori_loop(0, CHUNK, zrow, 0)

    def zacc(k, carry):
        pltpu.sync_copy(
            rows_a, acc_sh.at[pl.ds(sid * ROWS_PER_TILE + k * CHUNK, CHUNK)])
        return carry
    lax.fori_loop(0, ROWS_PER_TILE // CHUNK, zacc, 0)
    plsc.subcore_barrier()

    pltpu.async_copy(m_sp.at[sidx_v.at[0]], rows_a, sem_a)

    def body(i, carry):
        c0 = 2 * i
        c1 = c0 + 1
        pltpu.make_async_copy(m_sp.at[sidx_v.at[c0]], rows_a, sem_a).wait()
        pltpu.async_copy(m_sp.at[sidx_v.at[c1]], rows_b, sem_b)
        pltpu.sync_copy(rows_a, acc_sh.at[didx_v.at[c0]], add=True)
        pltpu.make_async_copy(m_sp.at[sidx_v.at[c1]], rows_b, sem_b).wait()

        @pl.when(i < NCHUNK // 2 - 1)
        def _():
            pltpu.async_copy(m_sp.at[sidx_v.at[c0 + 2]], rows_a, sem_a)
        pltpu.sync_copy(rows_b, acc_sh.at[didx_v.at[c1]], add=True)
        return carry
    lax.fori_loop(0, NCHUNK // 2, body, 0)
    plsc.subcore_barrier()

    def out(k, carry):
        roff = sid * ROWS_PER_TILE + k * CHUNK
        pltpu.sync_copy(acc_sh.at[pl.ds(roff, CHUNK)],
                        out_hbm.at[cid, pl.ds(roff, CHUNK)])
        return carry
    lax.fori_loop(0, ROWS_PER_TILE // CHUNK, out, 0)


@functools.partial(
    pl.kernel,
    out_type=jax.ShapeDtypeStruct((NW, NP), jnp.float32),
    mesh=_mesh(),
    scratch_types=[pltpu.VMEM((NP,), jnp.float32),
                   pltpu.VMEM((EPT,), jnp.int32),
                   pltpu.VMEM((EPT,), jnp.int32),
                   pltpu.VMEM((NP,), jnp.float32)],
    compiler_params=_SC_PARAMS,
)
def _edge1_call(src_hbm, dst_hbm, t_hbm, out_hbm, t_v, sidx_v, didx_v, acc_v):
    cid = lax.axis_index("c")
    sid = lax.axis_index("s")
    wid = sid * NC + cid
    base = wid * EPT
    pltpu.sync_copy(t_hbm, t_v)
    pltpu.sync_copy(src_hbm.at[pl.ds(base, EPT)], sidx_v)
    pltpu.sync_copy(dst_hbm.at[pl.ds(base, EPT)], didx_v)
    z = _Z16()

    def zero(j, carry):
        acc_v[pl.ds(j * 16, 16)] = z
        return carry
    lax.fori_loop(0, NP // 16, zero, 0)

    def body(j, carry):
        sl = pl.ds(j * 16, 16)
        sv = sidx_v[sl]
        dv = didx_v[sl]
        vals = plsc.load_gather(t_v, [sv])
        plsc.addupdate_scatter(acc_v, [dv], vals)
        return carry
    lax.fori_loop(0, EPT // 16, body, 0)
    pltpu.sync_copy(acc_v, out_hbm.at[wid])


def _prep_body(x_ref, w1_ref, degt_ref, m_ref, dinv_ref):
    deg = jnp.sum(degt_ref[...], axis=1, keepdims=True)
    row = lax.broadcasted_iota(jnp.int32, (NP, 1), 0)
    real = row < N
    deg = deg + jnp.where(real, 1.0, 0.0)
    dinv = jnp.where(real, lax.rsqrt(jnp.maximum(deg, 1e-30)), 0.0)
    h = jnp.dot(x_ref[...], w1_ref[...],
                preferred_element_type=jnp.float32,
                precision=lax.Precision.HIGHEST)
    m_ref[...] = dinv * h
    dinv_ref[...] = dinv


_prep_call = pl.pallas_call(
    _prep_body,
    out_shape=[jax.ShapeDtypeStruct((NP, DH), jnp.float32),
               jax.ShapeDtypeStruct((NP, 1), jnp.float32)],
)


def _mid_body(accp_ref, m_ref, dinv_ref, b1_ref, w2r_ref, t_ref):
    acc = accp_ref[0] + accp_ref[1] + m_ref[...]
    a = jnp.maximum(dinv_ref[...] * acc + b1_ref[...], 0.0)
    s = jnp.sum(a * w2r_ref[...], axis=1, keepdims=True)
    t_ref[...] = dinv_ref[...] * s


_mid_call = pl.pallas_call(
    _mid_body,
    out_shape=jax.ShapeDtypeStruct((NP, 1), jnp.float32),
)


def _final_body(l2t_ref, t_ref, dinv_ref, b2_ref, out_ref):
    es = jnp.sum(l2t_ref[...], axis=1, keepdims=True)
    out_ref[...] = dinv_ref[...] * (es + t_ref[...]) + b2_ref[...]


_final_call = pl.pallas_call(
    _final_body,
    out_shape=jax.ShapeDtypeStruct((NP, 1), jnp.float32),
)


def kernel(x, edge_index, node_id, W1, b1, W2, b2):
    src = edge_index[0]
    dst = edge_index[1]
    pad = jnp.full((EP - E,), N, jnp.int32)
    srcp = jnp.concatenate([src, pad])
    dstp = jnp.concatenate([dst, pad])
    src3 = srcp.reshape(NW, NCHUNK, CHUNK)
    dst3 = dstp.reshape(NW, NCHUNK, CHUNK)
    xp = jnp.pad(x, ((0, NP - N), (0, 0)))

    degp = _deg_call(dstp)
    m, dinv = _prep_call(xp, W1, degp.T)
    accp = _edge64_call(src3, dst3, m)
    t = _mid_call(accp, m, dinv,
                  b1.reshape(1, DH), W2.reshape(1, DH))
    l2 = _edge1_call(srcp, dstp, t.reshape(NP))
    out = _final_call(l2.T, t, dinv, b2.reshape(1, 1))
    return out[:N, 0]


# speedup vs baseline: 1.0604x; 1.0604x over previous
import functools

import jax
import jax.numpy as jnp
from jax import lax
from jax.experimental import pallas as pl
from jax.experimental.pallas import tpu as pltpu
from jax.experimental.pallas import tpu_sc as plsc

N = 10000
E = 320000
D_IN = 128
DH = 64

NC = 2
NS = 16
NW = NC * NS
CHUNK = 128
NCHUNK = 80
EPT = NCHUNK * CHUNK
EP = EPT * NW
NP = 12288
NPE = 10240
RPT_E = NPE // NS
EPS = E // NS
COLS_PER_TILE = NP // NW

_mesh = lambda: plsc.VectorSubcoreMesh(
    core_axis_name="c", subcore_axis_name="s", num_cores=NC, num_subcores=NS)

_SC_PARAMS = pltpu.CompilerParams(needs_layout_passes=False)
_SC_PARAMS_NT = pltpu.CompilerParams(needs_layout_passes=False,
                                     use_tc_tiling_on_sc=False)

_Z16 = lambda: jnp.zeros((16,), jnp.float32)


def _rsqrt16(d):
    y = plsc.bitcast(jnp.int32(0x5F3759DF) - (plsc.bitcast(d, jnp.int32) >> 1),
                     jnp.float32)
    for _ in range(3):
        y = y * (1.5 - 0.5 * d * y * y)
    return y


@functools.partial(
    pl.kernel,
    out_type=jax.ShapeDtypeStruct((NP,), jnp.float32),
    mesh=_mesh(),
    scratch_types=[pltpu.VMEM((EPS,), jnp.int32),
                   pltpu.VMEM((NP,), jnp.float32),
                   pltpu.VMEM((NS, COLS_PER_TILE), jnp.float32),
                   pltpu.VMEM((COLS_PER_TILE,), jnp.float32),
                   pltpu.VMEM_SHARED((NS, NP), jnp.float32)],
    compiler_params=_SC_PARAMS,
)
def _deg_call(dst_hbm, dinv_hbm, didx_v, acc_v, blk_v, res_v, stage_sp):
    cid = lax.axis_index("c")
    sid = lax.axis_index("s")
    pltpu.sync_copy(dst_hbm.at[pl.ds(sid * EPS, EPS)], didx_v)
    z = _Z16()

    def zero(j, carry):
        acc_v[pl.ds(j * 16, 16)] = z
        return carry
    lax.fori_loop(0, NP // 16, zero, 0)

    ones = jnp.ones((16,), jnp.float32)

    def body(j, carry):
        idx = didx_v[pl.ds(j * 16, 16)]
        plsc.addupdate_scatter(acc_v, [idx], ones)
        return carry
    lax.fori_loop(0, EPS // 16, body, 0)

    pltpu.sync_copy(acc_v, stage_sp.at[sid])
    plsc.subcore_barrier()

    colbase = cid * (NP // NC) + sid * COLS_PER_TILE
    pltpu.sync_copy(stage_sp.at[:, pl.ds(colbase, COLS_PER_TILE)], blk_v)
    lane = lax.iota(jnp.int32, 16)

    def comb(k, carry):
        sl = pl.ds(k * 16, 16)
        d = blk_v[0, sl]
        for s in range(1, NS):
            d = d + blk_v[s, sl]
        row = colbase + k * 16 + lane
        real = row < N
        d = d + 1.0
        y = _rsqrt16(d)
        res_v[sl] = jnp.where(real, y, 0.0)
        return carry
    lax.fori_loop(0, COLS_PER_TILE // 16, comb, 0)
    pltpu.sync_copy(res_v, dinv_hbm.at[pl.ds(colbase, COLS_PER_TILE)])


@functools.partial(
    pl.kernel,
    out_type=jax.ShapeDtypeStruct((NC, NPE, DH), jnp.float32),
    mesh=_mesh(),
    scratch_types=[pltpu.VMEM((NCHUNK, CHUNK), jnp.int32),
                   pltpu.VMEM((NCHUNK, CHUNK), jnp.int32),
                   pltpu.VMEM((CHUNK, DH), jnp.float32),
                   pltpu.VMEM((CHUNK, DH), jnp.float32),
                   pltpu.VMEM_SHARED((NPE, DH), jnp.float32),
                   pltpu.VMEM_SHARED((NPE, DH), jnp.float32),
                   pltpu.SemaphoreType.DMA,
                   pltpu.SemaphoreType.DMA,
                   pltpu.SemaphoreType.DMA],
    compiler_params=_SC_PARAMS_NT,
)
def _edge64_call(src_hbm, dst_hbm, m_hbm, out_hbm, sidx_v, didx_v, rows_a,
                 rows_b, acc_sh, m_sp, sem_a, sem_b, sem_c):
    cid = lax.axis_index("c")
    sid = lax.axis_index("s")
    wid = sid * NC + cid
    cp_s = pltpu.async_copy(src_hbm.at[wid], sidx_v, sem_a)
    cp_d = pltpu.async_copy(dst_hbm.at[wid], didx_v, sem_b)
    cp_m = pltpu.async_copy(
        m_hbm.at[pl.ds(sid * RPT_E, RPT_E)],
        m_sp.at[pl.ds(sid * RPT_E, RPT_E)], sem_c)
    z = _Z16()

    def zrow(r, carry):
        for c4 in range(DH // 16):
            rows_a[r, pl.ds(c4 * 16, 16)] = z
        return carry
    lax.fori_loop(0, CHUNK, zrow, 0)
    cp_s.wait()
    cp_d.wait()
    cp_m.wait()

    def zacc(k, carry):
        pltpu.sync_copy(
            rows_a, acc_sh.at[pl.ds(sid * RPT_E + k * CHUNK, CHUNK)])
        return carry
    lax.fori_loop(0, RPT_E // CHUNK, zacc, 0)
    plsc.subcore_barrier()

    pltpu.async_copy(m_sp.at[sidx_v.at[0]], rows_a, sem_a)

    def body(i, carry):
        c0 = 2 * i
        c1 = c0 + 1
        pltpu.make_async_copy(m_sp.at[sidx_v.at[c0]], rows_a, sem_a).wait()
        pltpu.async_copy(m_sp.at[sidx_v.at[c1]], rows_b, sem_b)
        pltpu.sync_copy(rows_a, acc_sh.at[didx_v.at[c0]], add=True)
        pltpu.make_async_copy(m_sp.at[sidx_v.at[c1]], rows_b, sem_b).wait()

        @pl.when(i < NCHUNK // 2 - 1)
        def _():
            pltpu.async_copy(m_sp.at[sidx_v.at[c0 + 2]], rows_a, sem_a)
        pltpu.sync_copy(rows_b, acc_sh.at[didx_v.at[c1]], add=True)
        return carry
    lax.fori_loop(0, NCHUNK // 2, body, 0)
    plsc.subcore_barrier()

    def out(k, carry):
        roff = sid * RPT_E + k * CHUNK
        pltpu.sync_copy(acc_sh.at[pl.ds(roff, CHUNK)],
                        out_hbm.at[cid, pl.ds(roff, CHUNK)])
        return carry
    lax.fori_loop(0, RPT_E // CHUNK, out, 0)


@functools.partial(
    pl.kernel,
    out_type=jax.ShapeDtypeStruct((NP,), jnp.float32),
    mesh=_mesh(),
    scratch_types=[pltpu.VMEM((NP,), jnp.float32),
                   pltpu.VMEM((EPS,), jnp.int32),
                   pltpu.VMEM((EPS,), jnp.int32),
                   pltpu.VMEM((NP,), jnp.float32),
                   pltpu.VMEM((NS, COLS_PER_TILE), jnp.float32),
                   pltpu.VMEM((COLS_PER_TILE,), jnp.float32),
                   pltpu.VMEM((COLS_PER_TILE,), jnp.float32),
                   pltpu.VMEM((16,), jnp.float32),
                   pltpu.VMEM_SHARED((NS, NP), jnp.float32)],
    compiler_params=_SC_PARAMS,
)
def _edge1f_call(src_hbm, dst_hbm, t_hbm, dinv_hbm, b2_hbm, out_hbm,
                 t_v, sidx_v, didx_v, acc_v, blk_v, dv_v, res_v, b2_v,
                 stage_sp):
    cid = lax.axis_index("c")
    sid = lax.axis_index("s")
    pltpu.sync_copy(t_hbm, t_v)
    pltpu.sync_copy(src_hbm.at[pl.ds(sid * EPS, EPS)], sidx_v)
    pltpu.sync_copy(dst_hbm.at[pl.ds(sid * EPS, EPS)], didx_v)
    pltpu.sync_copy(b2_hbm, b2_v)
    z = _Z16()

    def zero(j, carry):
        acc_v[pl.ds(j * 16, 16)] = z
        return carry
    lax.fori_loop(0, NP // 16, zero, 0)

    def body(j, carry):
        sl = pl.ds(j * 16, 16)
        sv = sidx_v[sl]
        dv = didx_v[sl]
        vals = plsc.load_gather(t_v, [sv])
        plsc.addupdate_scatter(acc_v, [dv], vals)
        return carry
    lax.fori_loop(0, EPS // 16, body, 0)

    pltpu.sync_copy(acc_v, stage_sp.at[sid])
    plsc.subcore_barrier()

    colbase = cid * (NP // NC) + sid * COLS_PER_TILE
    pltpu.sync_copy(stage_sp.at[:, pl.ds(colbase, COLS_PER_TILE)], blk_v)
    pltpu.sync_copy(dinv_hbm.at[pl.ds(colbase, COLS_PER_TILE)], dv_v)
    b2 = b2_v[pl.ds(0, 16)]

    def comb(k, carry):
        sl = pl.ds(k * 16, 16)
        es = blk_v[0, sl]
        for s in range(1, NS):
            es = es + blk_v[s, sl]
        tt = t_v[pl.ds(colbase + k * 16, 16)]
        res_v[sl] = dv_v[sl] * (es + tt) + b2
        return carry
    lax.fori_loop(0, COLS_PER_TILE // 16, comb, 0)
    pltpu.sync_copy(res_v, out_hbm.at[pl.ds(colbase, COLS_PER_TILE)])


def _prep_body(x_ref, w1_ref, dinv_ref, m_ref):
    h = jnp.dot(x_ref[...], w1_ref[...],
                preferred_element_type=jnp.float32,
                precision=lax.Precision.HIGHEST)
    hp = jnp.concatenate(
        [h, jnp.zeros((NP - N, DH), jnp.float32)], axis=0)
    m_ref[...] = dinv_ref[...] * hp


_prep_call = pl.pallas_call(
    _prep_body,
    out_shape=jax.ShapeDtypeStruct((NP, DH), jnp.float32),
)


def _mid_body(accp_ref, m_ref, dinv_ref, b1_ref, w2r_ref, t_ref):
    mf = lax.slice(m_ref[...], (0, 0), (NPE, DH))
    df = lax.slice(dinv_ref[...], (0, 0), (NPE, 1))
    acc = accp_ref[0] + accp_ref[1] + mf
    a = jnp.maximum(df * acc + b1_ref[...], 0.0)
    s = jnp.sum(a * w2r_ref[...], axis=1, keepdims=True)
    t_ref[...] = jnp.concatenate(
        [df * s, jnp.zeros((NP - NPE, 1), jnp.float32)], axis=0)


_mid_call = pl.pallas_call(
    _mid_body,
    out_shape=jax.ShapeDtypeStruct((NP, 1), jnp.float32),
)


def kernel(x, edge_index, node_id, W1, b1, W2, b2):
    src = edge_index[0]
    dst = edge_index[1]
    pad = jnp.full((EP - E,), N, jnp.int32)
    src3 = jnp.concatenate([src, pad]).reshape(NW, NCHUNK, CHUNK)
    dst3 = jnp.concatenate([dst, pad]).reshape(NW, NCHUNK, CHUNK)

    dinv = _deg_call(dst)
    dinv2 = dinv.reshape(NP, 1)
    m = _prep_call(x, W1, dinv2)
    accp = _edge64_call(src3, dst3, m)
    t = _mid_call(accp, m, dinv2,
                  b1.reshape(1, DH), W2.reshape(1, DH))
    out = _edge1f_call(src, dst, t.reshape(NP), dinv,
                       jnp.broadcast_to(b2, (16,)))
    return out[:N]


# speedup vs baseline: 1.0623x; 1.0018x over previous
import functools

import jax
import jax.numpy as jnp
from jax import lax
from jax.experimental import pallas as pl
from jax.experimental.pallas import tpu as pltpu
from jax.experimental.pallas import tpu_sc as plsc

N = 10000
E = 320000
D_IN = 128
DH = 64

NC = 2
NS = 16
NW = NC * NS
CHUNK = 128
NCHUNK = 80
EPT = NCHUNK * CHUNK
EP = EPT * NW
NP = 12288
NPE = 10240
RPT_E = NPE // NS
EPS = E // NS
COLS_PER_TILE = NP // NW

_mesh = lambda: plsc.VectorSubcoreMesh(
    core_axis_name="c", subcore_axis_name="s", num_cores=NC, num_subcores=NS)

_SC_PARAMS = pltpu.CompilerParams(needs_layout_passes=False)
_SC_PARAMS_NT = pltpu.CompilerParams(needs_layout_passes=False,
                                     use_tc_tiling_on_sc=False)

_Z16 = lambda: jnp.zeros((16,), jnp.float32)


def _rsqrt16(d):
    y = plsc.bitcast(jnp.int32(0x5F3759DF) - (plsc.bitcast(d, jnp.int32) >> 1),
                     jnp.float32)
    for _ in range(3):
        y = y * (1.5 - 0.5 * d * y * y)
    return y


@functools.partial(
    pl.kernel,
    out_type=jax.ShapeDtypeStruct((NP,), jnp.float32),
    mesh=_mesh(),
    scratch_types=[pltpu.VMEM((EPS,), jnp.int32),
                   pltpu.VMEM((NP,), jnp.float32),
                   pltpu.VMEM((NS, COLS_PER_TILE), jnp.float32),
                   pltpu.VMEM((COLS_PER_TILE,), jnp.float32),
                   pltpu.VMEM_SHARED((NS, NP), jnp.float32)],
    compiler_params=_SC_PARAMS,
)
def _deg_call(dst_hbm, dinv_hbm, didx_v, acc_v, blk_v, res_v, stage_sp):
    cid = lax.axis_index("c")
    sid = lax.axis_index("s")
    pltpu.sync_copy(dst_hbm.at[pl.ds(sid * EPS, EPS)], didx_v)
    z = _Z16()

    def zero(j, carry):
        acc_v[pl.ds(j * 16, 16)] = z
        return carry
    lax.fori_loop(0, NP // 16, zero, 0)

    ones = jnp.ones((16,), jnp.float32)

    def body(j, carry):
        idx = didx_v[pl.ds(j * 16, 16)]
        plsc.addupdate_scatter(acc_v, [idx], ones)
        return carry
    lax.fori_loop(0, EPS // 16, body, 0)

    pltpu.sync_copy(acc_v, stage_sp.at[sid])
    plsc.subcore_barrier()

    colbase = cid * (NP // NC) + sid * COLS_PER_TILE
    pltpu.sync_copy(stage_sp.at[:, pl.ds(colbase, COLS_PER_TILE)], blk_v)
    lane = lax.iota(jnp.int32, 16)

    def comb(k, carry):
        sl = pl.ds(k * 16, 16)
        d = blk_v[0, sl]
        for s in range(1, NS):
            d = d + blk_v[s, sl]
        row = colbase + k * 16 + lane
        real = row < N
        d = d + 1.0
        y = _rsqrt16(d)
        res_v[sl] = jnp.where(real, y, 0.0)
        return carry
    lax.fori_loop(0, COLS_PER_TILE // 16, comb, 0)
    pltpu.sync_copy(res_v, dinv_hbm.at[pl.ds(colbase, COLS_PER_TILE)])


@functools.partial(
    pl.kernel,
    out_type=jax.ShapeDtypeStruct((NC, NPE, DH), jnp.float32),
    mesh=_mesh(),
    scratch_types=[pltpu.VMEM((NCHUNK, CHUNK), jnp.int32),
                   pltpu.VMEM((NCHUNK, CHUNK), jnp.int32),
                   pltpu.VMEM((CHUNK, DH), jnp.float32),
                   pltpu.VMEM((CHUNK, DH), jnp.float32),
                   pltpu.VMEM_SHARED((NPE, DH), jnp.float32),
                   pltpu.VMEM_SHARED((NPE, DH), jnp.float32),
                   pltpu.SemaphoreType.DMA,
                   pltpu.SemaphoreType.DMA,
                   pltpu.SemaphoreType.DMA,
                   pltpu.SemaphoreType.DMA,
                   pltpu.SemaphoreType.DMA],
    compiler_params=_SC_PARAMS_NT,
)
def _edge64_call(src_hbm, dst_hbm, m_hbm, out_hbm, sidx_v, didx_v, rows_a,
                 rows_b, acc_sh, m_sp, sem_a, sem_b, sem_c, sem_sa, sem_sb):
    cid = lax.axis_index("c")
    sid = lax.axis_index("s")
    wid = sid * NC + cid
    cp_s = pltpu.async_copy(src_hbm.at[wid], sidx_v, sem_a)
    cp_d = pltpu.async_copy(dst_hbm.at[wid], didx_v, sem_b)
    cp_m = pltpu.async_copy(
        m_hbm.at[pl.ds(sid * RPT_E, RPT_E)],
        m_sp.at[pl.ds(sid * RPT_E, RPT_E)], sem_c)
    z = _Z16()

    def zrow(r, carry):
        for c4 in range(DH // 16):
            rows_a[r, pl.ds(c4 * 16, 16)] = z
        return carry
    lax.fori_loop(0, CHUNK, zrow, 0)
    cp_s.wait()
    cp_d.wait()
    cp_m.wait()

    def zacc(k, carry):
        pltpu.sync_copy(
            rows_a, acc_sh.at[pl.ds(sid * RPT_E + k * CHUNK, CHUNK)])
        return carry
    lax.fori_loop(0, RPT_E // CHUNK, zacc, 0)
    plsc.subcore_barrier()

    pltpu.async_copy(m_sp.at[sidx_v.at[0]], rows_a, sem_a)
    NH = NCHUNK // 2

    def body(i, carry):
        c0 = 2 * i
        c1 = c0 + 1
        pltpu.make_async_copy(m_sp.at[sidx_v.at[c0]], rows_a, sem_a).wait()

        @pl.when(i > 0)
        def _():
            pltpu.make_async_copy(rows_b, acc_sh.at[didx_v.at[c1]],
                                  sem_sb).wait()
        pltpu.async_copy(m_sp.at[sidx_v.at[c1]], rows_b, sem_b)
        d_sa = pltpu.async_copy(rows_a, acc_sh.at[didx_v.at[c0]], sem_sa,
                                add=True)
        pltpu.make_async_copy(m_sp.at[sidx_v.at[c1]], rows_b, sem_b).wait()

        @pl.when(i < NH - 1)
        def _():
            d_sa.wait()
            pltpu.async_copy(m_sp.at[sidx_v.at[c0 + 2]], rows_a, sem_a)
        pltpu.async_copy(rows_b, acc_sh.at[didx_v.at[c1]], sem_sb, add=True)
        return carry
    lax.fori_loop(0, NH, body, 0)
    pltpu.make_async_copy(rows_a, acc_sh.at[didx_v.at[0]], sem_sa).wait()
    pltpu.make_async_copy(rows_b, acc_sh.at[didx_v.at[0]], sem_sb).wait()
    plsc.subcore_barrier()

    def out(k, carry):
        roff = sid * RPT_E + k * CHUNK
        pltpu.sync_copy(acc_sh.at[pl.ds(roff, CHUNK)],
                        out_hbm.at[cid, pl.ds(roff, CHUNK)])
        return carry
    lax.fori_loop(0, RPT_E // CHUNK, out, 0)


@functools.partial(
    pl.kernel,
    out_type=jax.ShapeDtypeStruct((NP,), jnp.float32),
    mesh=_mesh(),
    scratch_types=[pltpu.VMEM((NP,), jnp.float32),
                   pltpu.VMEM((EPS,), jnp.int32),
                   pltpu.VMEM((EPS,), jnp.int32),
                   pltpu.VMEM((NP,), jnp.float32),
                   pltpu.VMEM((NS, COLS_PER_TILE), jnp.float32),
                   pltpu.VMEM((COLS_PER_TILE,), jnp.float32),
                   pltpu.VMEM((COLS_PER_TILE,), jnp.float32),
                   pltpu.VMEM((16,), jnp.float32),
                   pltpu.VMEM_SHARED((NS, NP), jnp.float32)],
    compiler_params=_SC_PARAMS,
)
def _edge1f_call(src_hbm, dst_hbm, t_hbm, dinv_hbm, b2_hbm, out_hbm,
                 t_v, sidx_v, didx_v, acc_v, blk_v, dv_v, res_v, b2_v,
                 stage_sp):
    cid = lax.axis_index("c")
    sid = lax.axis_index("s")
    pltpu.sync_copy(t_hbm, t_v)
    pltpu.sync_copy(src_hbm.at[pl.ds(sid * EPS, EPS)], sidx_v)
    pltpu.sync_copy(dst_hbm.at[pl.ds(sid * EPS, EPS)], didx_v)
    pltpu.sync_copy(b2_hbm, b2_v)
    z = _Z16()

    def zero(j, carry):
        acc_v[pl.ds(j * 16, 16)] = z
        return carry
    lax.fori_loop(0, NP // 16, zero, 0)

    def body(j, carry):
        sl = pl.ds(j * 16, 16)
        sv = sidx_v[sl]
        dv = didx_v[sl]
        vals = plsc.load_gather(t_v, [sv])
        plsc.addupdate_scatter(acc_v, [dv], vals)
        return carry
    lax.fori_loop(0, EPS // 16, body, 0)

    pltpu.sync_copy(acc_v, stage_sp.at[sid])
    plsc.subcore_barrier()

    colbase = cid * (NP // NC) + sid * COLS_PER_TILE
    pltpu.sync_copy(stage_sp.at[:, pl.ds(colbase, COLS_PER_TILE)], blk_v)
    pltpu.sync_copy(dinv_hbm.at[pl.ds(colbase, COLS_PER_TILE)], dv_v)
    b2 = b2_v[pl.ds(0, 16)]

    def comb(k, carry):
        sl = pl.ds(k * 16, 16)
        es = blk_v[0, sl]
        for s in range(1, NS):
            es = es + blk_v[s, sl]
        tt = t_v[pl.ds(colbase + k * 16, 16)]
        res_v[sl] = dv_v[sl] * (es + tt) + b2
        return carry
    lax.fori_loop(0, COLS_PER_TILE // 16, comb, 0)
    pltpu.sync_copy(res_v, out_hbm.at[pl.ds(colbase, COLS_PER_TILE)])


def _prep_body(x_ref, w1_ref, dinv_ref, m_ref):
    h = jnp.dot(x_ref[...], w1_ref[...],
                preferred_element_type=jnp.float32,
                precision=lax.Precision.HIGHEST)
    hp = jnp.concatenate(
        [h, jnp.zeros((NP - N, DH), jnp.float32)], axis=0)
    m_ref[...] = dinv_ref[...] * hp


_prep_call = pl.pallas_call(
    _prep_body,
    out_shape=jax.ShapeDtypeStruct((NP, DH), jnp.float32),
)


def _mid_body(accp_ref, m_ref, dinv_ref, b1_ref, w2r_ref, t_ref):
    mf = lax.slice(m_ref[...], (0, 0), (NPE, DH))
    df = lax.slice(dinv_ref[...], (0, 0), (NPE, 1))
    acc = accp_ref[0] + accp_ref[1] + mf
    a = jnp.maximum(df * acc + b1_ref[...], 0.0)
    s = jnp.sum(a * w2r_ref[...], axis=1, keepdims=True)
    t_ref[...] = jnp.concatenate(
        [df * s, jnp.zeros((NP - NPE, 1), jnp.float32)], axis=0)


_mid_call = pl.pallas_call(
    _mid_body,
    out_shape=jax.ShapeDtypeStruct((NP, 1), jnp.float32),
)


def kernel(x, edge_index, node_id, W1, b1, W2, b2):
    src = edge_index[0]
    dst = edge_index[1]
    pad = jnp.full((EP - E,), N, jnp.int32)
    src3 = jnp.concatenate([src, pad]).reshape(NW, NCHUNK, CHUNK)
    dst3 = jnp.concatenate([dst, pad]).reshape(NW, NCHUNK, CHUNK)

    dinv = _deg_call(dst)
    dinv2 = dinv.reshape(NP, 1)
    m = _prep_call(x, W1, dinv2)
    accp = _edge64_call(src3, dst3, m)
    t = _mid_call(accp, m, dinv2,
                  b1.reshape(1, DH), W2.reshape(1, DH))
    out = _edge1f_call(src, dst, t.reshape(NP), dinv,
                       jnp.broadcast_to(b2, (16,)))
    return out[:N]
